# Initial kernel scaffold; baseline (speedup 1.0000x reference)
import jax
import jax.numpy as jnp
from jax.experimental import pallas as pl


def kernel(x, edge_index, layer, Wl_stack, Wr_stack, b_stack, W2_l, W2_r, b2):
    raise NotImplementedError("write your pallas kernel here")


# speedup vs baseline: 7.3513x; 7.3513x over previous
import functools

import jax
import jax.numpy as jnp
from jax import lax
from jax.experimental import pallas as pl
from jax.experimental.pallas import tpu as pltpu
from jax.experimental.pallas import tpu_sc as plsc

N = 10000
NP = 10240
D = 128
E = 320000
NC = 2
NS = 16
NW = NC * NS
CS = 80
WCH = 25
NWIN = 5
RPS = NP // NS
BN = 1024

_mesh = plsc.VectorSubcoreMesh(core_axis_name="c", subcore_axis_name="s")


def _make_agg(compute_deg):
  out_type = [jax.ShapeDtypeStruct((NC, NP, D), jnp.float32)]
  scratch = [
      pltpu.VMEM((WCH, CS), jnp.int32),
      pltpu.VMEM((WCH, CS), jnp.int32),
      pltpu.VMEM((CS, D), jnp.float32),
      pltpu.VMEM_SHARED((NP, D), jnp.float32),
      pltpu.SemaphoreType.DMA,
  ]
  if compute_deg:
    out_type.append(jax.ShapeDtypeStruct((NW, NP), jnp.float32))
    scratch.insert(3, pltpu.VMEM((NP,), jnp.float32))

  @functools.partial(pl.kernel, mesh=_mesh, out_type=out_type,
                     scratch_types=scratch)
  def agg(*refs):
    if compute_deg:
      (x_hbm, src_hbm, dst_hbm, aggr_out, deg_out,
       src_v, dst_v, rows_v, deg_v, acc_sh, sem) = refs
    else:
      (x_hbm, src_hbm, dst_hbm, aggr_out,
       src_v, dst_v, rows_v, acc_sh, sem) = refs

    c = lax.axis_index("c")
    s = lax.axis_index("s")
    w = c * NS + s

    z16 = jnp.zeros((16,), jnp.float32)

    def zrow(i, carry):
      for l in range(D // 16):
        rows_v[i, pl.ds(l * 16, 16)] = z16
      return carry
    lax.fori_loop(0, CS, zrow, 0)
    for t in range(RPS // CS):
      pltpu.sync_copy(rows_v, acc_sh.at[pl.ds(s * RPS + t * CS, CS)])
    if compute_deg:
      def zdeg(i, carry):
        deg_v[pl.ds(i * 16, 16)] = z16
        return carry
      lax.fori_loop(0, NP // 16, zdeg, 0)
    plsc.subcore_barrier()

    onehot = jnp.where(lax.iota(jnp.int32, 16) == 0, 1.0, 0.0)

    def window(t, carry):
      pltpu.sync_copy(src_hbm.at[w, t], src_v)
      pltpu.sync_copy(dst_hbm.at[w, t], dst_v)

      def chunk(j, carry2):
        cp = pltpu.async_copy(x_hbm.at[src_v.at[j]], rows_v, sem)
        if compute_deg:
          for l in range(CS // 16):
            dvec = dst_v[j, pl.ds(l * 16, 16)]
            for q in range(16):
              d = dvec[q]
              deg_v[pl.ds(d, 16)] = deg_v[pl.ds(d, 16)] + onehot
        cp.wait()
        pltpu.sync_copy(rows_v, acc_sh.at[dst_v.at[j]], add=True)
        return carry2
      lax.fori_loop(0, WCH, chunk, 0)
      return carry
    lax.fori_loop(0, NWIN, window, 0)

    plsc.subcore_barrier()
    pltpu.sync_copy(acc_sh.at[pl.ds(s * RPS, RPS)],
                    aggr_out.at[c, pl.ds(s * RPS, RPS)])
    if compute_deg:
      pltpu.sync_copy(deg_v, deg_out.at[w])

  return agg


_agg_with_deg = _make_agg(True)
_agg_no_deg = _make_agg(False)


def _mm_body(relu, aggrp_ref, deg_ref, x_ref, wl_ref, wr_ref, b_ref, out_ref):
  a = aggrp_ref[0] + aggrp_ref[1]
  deg = jnp.sum(deg_ref[...], axis=0)
  inv = 1.0 / jnp.maximum(deg, 1.0)
  m = a * inv[:, None]
  acc = jnp.dot(m, wl_ref[...], preferred_element_type=jnp.float32)
  acc = acc + jnp.dot(x_ref[...], wr_ref[...],
                      preferred_element_type=jnp.float32)
  acc = acc + b_ref[...]
  if relu:
    acc = jnp.maximum(acc, 0.0)
  out_ref[...] = acc


def _mm(aggrp, degp, x, wl, wr, b, relu):
  grid = (pl.cdiv(N, BN),)
  return pl.pallas_call(
      functools.partial(_mm_body, relu),
      grid=grid,
      in_specs=[
          pl.BlockSpec((NC, BN, D), lambda i: (0, i, 0)),
          pl.BlockSpec((NW, BN), lambda i: (0, i)),
          pl.BlockSpec((BN, D), lambda i: (i, 0)),
          pl.BlockSpec((D, D), lambda i: (0, 0)),
          pl.BlockSpec((D, D), lambda i: (0, 0)),
          pl.BlockSpec((1, D), lambda i: (0, 0)),
      ],
      out_specs=pl.BlockSpec((BN, D), lambda i: (i, 0)),
      out_shape=jax.ShapeDtypeStruct((N, D), jnp.float32),
  )(aggrp, degp, x, wl, wr, b)


def kernel(x, edge_index, layer, Wl_stack, Wr_stack, b_stack, W2_l, W2_r, b2):
  src = edge_index[0].astype(jnp.int32).reshape(NW, NWIN, WCH, CS)
  dst = edge_index[1].astype(jnp.int32).reshape(NW, NWIN, WCH, CS)
  Wl = Wl_stack[layer]
  Wr = Wr_stack[layer]
  b = b_stack[layer]

  aggr1, degp = _agg_with_deg(x, src, dst)
  h = _mm(aggr1, degp, x, Wl, Wr, b.reshape(1, D), relu=True)
  (aggr2,) = _agg_no_deg(h, src, dst)
  out = _mm(aggr2, degp, h, W2_l, W2_r, b2.reshape(1, D), relu=False)
  return out


# speedup vs baseline: 9.1756x; 1.2482x over previous
import functools

import jax
import jax.numpy as jnp
from jax import lax
from jax.experimental import pallas as pl
from jax.experimental.pallas import tpu as pltpu
from jax.experimental.pallas import tpu_sc as plsc

N = 10000
NP = 10240
D = 128
E = 320000
NC = 2
NS = 16
NW = NC * NS
CS = 80
WCH = 25
NWIN = 5
RPS = NP // NS
BN = 1024

_mesh = plsc.VectorSubcoreMesh(core_axis_name="c", subcore_axis_name="s")


def _make_agg(compute_deg):
  out_type = [jax.ShapeDtypeStruct((NC, NP, D), jnp.float32)]
  scratch = [
      pltpu.VMEM((WCH, CS), jnp.int32),
      pltpu.VMEM((WCH, CS), jnp.int32),
      pltpu.VMEM((2, CS, D), jnp.float32),
      pltpu.VMEM_SHARED((NP, D), jnp.float32),
      pltpu.SemaphoreType.DMA,
      pltpu.SemaphoreType.DMA,
  ]
  if compute_deg:
    out_type.append(jax.ShapeDtypeStruct((NW, NP), jnp.float32))
    scratch.insert(3, pltpu.VMEM((NP,), jnp.float32))

  @functools.partial(pl.kernel, mesh=_mesh, out_type=out_type,
                     scratch_types=scratch)
  def agg(*refs):
    if compute_deg:
      (x_hbm, src_hbm, dst_hbm, aggr_out, deg_out,
       src_v, dst_v, rows_v, deg_v, acc_sh, sem_g, sem_s) = refs
    else:
      (x_hbm, src_hbm, dst_hbm, aggr_out,
       src_v, dst_v, rows_v, acc_sh, sem_g, sem_s) = refs

    c = lax.axis_index("c")
    s = lax.axis_index("s")
    w = c * NS + s

    z16 = jnp.zeros((16,), jnp.float32)

    def zrow(i, carry):
      for l in range(D // 16):
        rows_v[0, i, pl.ds(l * 16, 16)] = z16
      return carry
    lax.fori_loop(0, CS, zrow, 0)
    for t in range(RPS // CS):
      pltpu.sync_copy(rows_v.at[0], acc_sh.at[pl.ds(s * RPS + t * CS, CS)])
    if compute_deg:
      def zdeg(i, carry):
        deg_v[pl.ds(i * 16, 16)] = z16
        return carry
      lax.fori_loop(0, NP // 16, zdeg, 0)
    plsc.subcore_barrier()

    onehot = jnp.where(lax.iota(jnp.int32, 16) == 0, 1.0, 0.0)

    def hist(j):
      if compute_deg:
        for l in range(CS // 16):
          dvec = dst_v[j, pl.ds(l * 16, 16)]
          for q in range(16):
            d = dvec[q]
            deg_v[pl.ds(d, 16)] = deg_v[pl.ds(d, 16)] + onehot

    def window(t, carry):
      pltpu.sync_copy(src_hbm.at[w, t], src_v)
      pltpu.sync_copy(dst_hbm.at[w, t], dst_v)
      pltpu.async_copy(x_hbm.at[src_v.at[0]], rows_v.at[0], sem_g).wait()

      def chunk(j, carry2):
        p = lax.rem(j, 2)
        cpg = pltpu.async_copy(x_hbm.at[src_v.at[j + 1]], rows_v.at[1 - p],
                               sem_g)
        cps = pltpu.async_copy(rows_v.at[p], acc_sh.at[dst_v.at[j]], sem_s,
                               add=True)
        hist(j)
        cps.wait()
        cpg.wait()
        return carry2
      lax.fori_loop(0, WCH - 1, chunk, 0)
      pltpu.async_copy(rows_v.at[(WCH - 1) % 2], acc_sh.at[dst_v.at[WCH - 1]],
                       sem_s, add=True).wait()
      hist(WCH - 1)
      return carry
    lax.fori_loop(0, NWIN, window, 0)

    plsc.subcore_barrier()
    pltpu.sync_copy(acc_sh.at[pl.ds(s * RPS, RPS)],
                    aggr_out.at[c, pl.ds(s * RPS, RPS)])
    if compute_deg:
      pltpu.sync_copy(deg_v, deg_out.at[w])

  return agg


_agg_with_deg = _make_agg(True)
_agg_no_deg = _make_agg(False)


def _mm_body(relu, aggrp_ref, deg_ref, x_ref, wl_ref, wr_ref, b_ref, out_ref):
  a = aggrp_ref[0] + aggrp_ref[1]
  deg = jnp.sum(deg_ref[...], axis=0)
  inv = 1.0 / jnp.maximum(deg, 1.0)
  m = a * inv[:, None]
  acc = jnp.dot(m, wl_ref[...], preferred_element_type=jnp.float32)
  acc = acc + jnp.dot(x_ref[...], wr_ref[...],
                      preferred_element_type=jnp.float32)
  acc = acc + b_ref[...]
  if relu:
    acc = jnp.maximum(acc, 0.0)
  out_ref[...] = acc


def _mm(aggrp, degp, x, wl, wr, b, relu):
  grid = (pl.cdiv(N, BN),)
  return pl.pallas_call(
      functools.partial(_mm_body, relu),
      grid=grid,
      in_specs=[
          pl.BlockSpec((NC, BN, D), lambda i: (0, i, 0)),
          pl.BlockSpec((NW, BN), lambda i: (0, i)),
          pl.BlockSpec((BN, D), lambda i: (i, 0)),
          pl.BlockSpec((D, D), lambda i: (0, 0)),
          pl.BlockSpec((D, D), lambda i: (0, 0)),
          pl.BlockSpec((1, D), lambda i: (0, 0)),
      ],
      out_specs=pl.BlockSpec((BN, D), lambda i: (i, 0)),
      out_shape=jax.ShapeDtypeStruct((N, D), jnp.float32),
  )(aggrp, degp, x, wl, wr, b)


def kernel(x, edge_index, layer, Wl_stack, Wr_stack, b_stack, W2_l, W2_r, b2):
  src = edge_index[0].astype(jnp.int32).reshape(NW, NWIN, WCH, CS)
  dst = edge_index[1].astype(jnp.int32).reshape(NW, NWIN, WCH, CS)
  Wl = Wl_stack[layer]
  Wr = Wr_stack[layer]
  b = b_stack[layer]

  aggr1, degp = _agg_with_deg(x, src, dst)
  h = _mm(aggr1, degp, x, Wl, Wr, b.reshape(1, D), relu=True)
  (aggr2,) = _agg_no_deg(h, src, dst)
  out = _mm(aggr2, degp, h, W2_l, W2_r, b2.reshape(1, D), relu=False)
  return out


# speedup vs baseline: 9.2292x; 1.0058x over previous
import functools

import jax
import jax.numpy as jnp
from jax import lax
from jax.experimental import pallas as pl
from jax.experimental.pallas import tpu as pltpu
from jax.experimental.pallas import tpu_sc as plsc

N = 10000
NP = 10240
D = 128
E = 320000
NC = 2
NS = 16
NW = NC * NS
CS = 80
WCH = 25
NWIN = 5
RPS = NP // NS
BN = 1024

_mesh = plsc.VectorSubcoreMesh(core_axis_name="c", subcore_axis_name="s")


def _make_agg(compute_deg):
  out_type = [jax.ShapeDtypeStruct((NC, NP, D), jnp.float32)]
  scratch = [
      pltpu.VMEM((WCH, CS), jnp.int32),
      pltpu.VMEM((WCH, CS), jnp.int32),
      pltpu.VMEM((3, CS, D), jnp.float32),
      pltpu.VMEM_SHARED((NP, D), jnp.float32),
      pltpu.SemaphoreType.DMA,
      pltpu.SemaphoreType.DMA,
  ]
  if compute_deg:
    out_type.append(jax.ShapeDtypeStruct((NW, NP), jnp.float32))
    scratch.insert(3, pltpu.VMEM((NP,), jnp.float32))

  @functools.partial(pl.kernel, mesh=_mesh, out_type=out_type,
                     scratch_types=scratch)
  def agg(*refs):
    if compute_deg:
      (x_hbm, src_hbm, dst_hbm, aggr_out, deg_out,
       src_v, dst_v, rows_v, deg_v, acc_sh, sem_g, sem_s) = refs
    else:
      (x_hbm, src_hbm, dst_hbm, aggr_out,
       src_v, dst_v, rows_v, acc_sh, sem_g, sem_s) = refs

    c = lax.axis_index("c")
    s = lax.axis_index("s")
    w = c * NS + s

    z16 = jnp.zeros((16,), jnp.float32)

    def zrow(i, carry):
      for l in range(D // 16):
        rows_v[0, i, pl.ds(l * 16, 16)] = z16
      return carry
    lax.fori_loop(0, CS, zrow, 0)
    for t in range(RPS // CS):
      pltpu.sync_copy(rows_v.at[0], acc_sh.at[pl.ds(s * RPS + t * CS, CS)])
    if compute_deg:
      def zdeg(i, carry):
        deg_v[pl.ds(i * 16, 16)] = z16
        return carry
      lax.fori_loop(0, NP // 16, zdeg, 0)
    plsc.subcore_barrier()

    onehot = jnp.where(lax.iota(jnp.int32, 16) == 0, 1.0, 0.0)

    def hist(j):
      if compute_deg:
        for l in range(CS // 16):
          dvec = dst_v[j, pl.ds(l * 16, 16)]
          for q in range(16):
            d = dvec[q]
            deg_v[pl.ds(d, 16)] = deg_v[pl.ds(d, 16)] + onehot

    def drain_g():
      pltpu.make_async_copy(x_hbm.at[src_v.at[0]], rows_v.at[0], sem_g).wait()

    def drain_s():
      pltpu.make_async_copy(x_hbm.at[src_v.at[0]], rows_v.at[0], sem_s).wait()

    def window(t, carry):
      pltpu.sync_copy(src_hbm.at[w, t], src_v)
      pltpu.sync_copy(dst_hbm.at[w, t], dst_v)
      pltpu.async_copy(x_hbm.at[src_v.at[0]], rows_v.at[0], sem_g)

      def chunk(j, carry2):
        p = lax.rem(j, 3)
        drain_g()
        pltpu.async_copy(x_hbm.at[src_v.at[j + 1]],
                         rows_v.at[lax.rem(j + 1, 3)], sem_g)
        pltpu.async_copy(rows_v.at[p], acc_sh.at[dst_v.at[j]], sem_s,
                         add=True)
        hist(j)

        @pl.when(j > 0)
        def _():
          drain_s()
        return carry2
      lax.fori_loop(0, WCH - 1, chunk, 0)
      drain_g()
      pltpu.async_copy(rows_v.at[(WCH - 1) % 3], acc_sh.at[dst_v.at[WCH - 1]],
                       sem_s, add=True)
      hist(WCH - 1)
      drain_s()
      drain_s()
      return carry
    lax.fori_loop(0, NWIN, window, 0)

    plsc.subcore_barrier()
    pltpu.sync_copy(acc_sh.at[pl.ds(s * RPS, RPS)],
                    aggr_out.at[c, pl.ds(s * RPS, RPS)])
    if compute_deg:
      pltpu.sync_copy(deg_v, deg_out.at[w])

  return agg


_agg_with_deg = _make_agg(True)
_agg_no_deg = _make_agg(False)


def _mm_body(relu, aggrp_ref, deg_ref, x_ref, wl_ref, wr_ref, b_ref, out_ref):
  a = aggrp_ref[0] + aggrp_ref[1]
  deg = jnp.sum(deg_ref[...], axis=0)
  inv = 1.0 / jnp.maximum(deg, 1.0)
  m = a * inv[:, None]
  acc = jnp.dot(m, wl_ref[...], preferred_element_type=jnp.float32)
  acc = acc + jnp.dot(x_ref[...], wr_ref[...],
                      preferred_element_type=jnp.float32)
  acc = acc + b_ref[...]
  if relu:
    acc = jnp.maximum(acc, 0.0)
  out_ref[...] = acc


def _mm(aggrp, degp, x, wl, wr, b, relu):
  grid = (pl.cdiv(N, BN),)
  return pl.pallas_call(
      functools.partial(_mm_body, relu),
      grid=grid,
      in_specs=[
          pl.BlockSpec((NC, BN, D), lambda i: (0, i, 0)),
          pl.BlockSpec((NW, BN), lambda i: (0, i)),
          pl.BlockSpec((BN, D), lambda i: (i, 0)),
          pl.BlockSpec((D, D), lambda i: (0, 0)),
          pl.BlockSpec((D, D), lambda i: (0, 0)),
          pl.BlockSpec((1, D), lambda i: (0, 0)),
      ],
      out_specs=pl.BlockSpec((BN, D), lambda i: (i, 0)),
      out_shape=jax.ShapeDtypeStruct((N, D), jnp.float32),
  )(aggrp, degp, x, wl, wr, b)


def kernel(x, edge_index, layer, Wl_stack, Wr_stack, b_stack, W2_l, W2_r, b2):
  src = edge_index[0].astype(jnp.int32).reshape(NW, NWIN, WCH, CS)
  dst = edge_index[1].astype(jnp.int32).reshape(NW, NWIN, WCH, CS)
  Wl = Wl_stack[layer]
  Wr = Wr_stack[layer]
  b = b_stack[layer]

  aggr1, degp = _agg_with_deg(x, src, dst)
  h = _mm(aggr1, degp, x, Wl, Wr, b.reshape(1, D), relu=True)
  (aggr2,) = _agg_no_deg(h, src, dst)
  out = _mm(aggr2, degp, h, W2_l, W2_r, b2.reshape(1, D), relu=False)
  return out
